# baseline (speedup 1.0000x reference)
import functools

import jax
import jax.numpy as jnp
from jax import lax
from jax.experimental import pallas as pl
from jax.experimental.pallas import tpu as pltpu
from jax.experimental.pallas import tpu_sc as plsc

DIM = 4096
N_TOKENS = 8192
L = 16
NC = 2
NS = 16
NW = NC * NS
ROWS_PER_W = N_TOKENS // NW
R = 4
N_CHUNKS = ROWS_PER_W // R
NBLK = DIM // L


def _permute_body(x_hbm, p_hbm, out_hbm, p_v, x0, x1, o0, o1,
                  si0, si1, so0, so1):
    c = lax.axis_index("c")
    s = lax.axis_index("s")
    wid = s * NC + c
    row0 = wid * ROWS_PER_W

    xb = (x0, x1)
    ob = (o0, o1)
    sin = (si0, si1)
    sout = (so0, so1)

    pltpu.sync_copy(p_hbm, p_v)

    def in_copy(g, b):
        return pltpu.make_async_copy(
            x_hbm.at[pl.ds(row0 + g * R, R)], xb[b], sin[b])

    def out_copy(g, b):
        return pltpu.make_async_copy(
            ob[b], out_hbm.at[pl.ds(row0 + g * R, R)], sout[b])

    in_copy(0, 0).start()

    def loop(i, carry):
        for b in range(2):
            g = i * 2 + b

            @pl.when(g + 1 < N_CHUNKS)
            def _():
                in_copy(g + 1, 1 - b).start()

            in_copy(g, b).wait()

            @pl.when(g >= 2)
            def _():
                out_copy(g - 2, b).wait()

            x_v = xb[b]
            o_v = ob[b]

            @plsc.parallel_loop(0, NBLK, unroll=4)
            def blk(jb):
                idx = p_v[pl.ds(jb * L, L)]
                for r in range(R):
                    ridx = jnp.full((L,), r, dtype=jnp.int32)
                    vals = plsc.load_gather(x_v, [ridx, idx])
                    o_v[r, pl.ds(jb * L, L)] = vals

            out_copy(g, b).start()
        return carry

    lax.fori_loop(0, N_CHUNKS // 2, loop, 0)

    for b in range(2):
        out_copy(N_CHUNKS - 2 + b, b).wait()


@jax.jit
def _permute_in(x, p32):
    mesh = plsc.VectorSubcoreMesh(core_axis_name="c", subcore_axis_name="s")
    f = functools.partial(
        pl.kernel,
        out_type=jax.ShapeDtypeStruct((N_TOKENS, DIM), jnp.float32),
        mesh=mesh,
        scratch_types=[
            pltpu.VMEM((DIM,), jnp.int32),
            pltpu.VMEM((R, DIM), jnp.float32),
            pltpu.VMEM((R, DIM), jnp.float32),
            pltpu.VMEM((R, DIM), jnp.float32),
            pltpu.VMEM((R, DIM), jnp.float32),
            pltpu.SemaphoreType.DMA,
            pltpu.SemaphoreType.DMA,
            pltpu.SemaphoreType.DMA,
            pltpu.SemaphoreType.DMA,
        ],
        compiler_params=pltpu.CompilerParams(
            use_tc_tiling_on_sc=False, needs_layout_passes=False
        ),
    )(_permute_body)
    return f(x, p32)


def kernel(x, permute):
    return _permute_in(x, permute.astype(jnp.int32))


# speedup vs baseline: 2.9096x; 2.9096x over previous
import functools

import jax
import jax.numpy as jnp
from jax import lax
from jax.experimental import pallas as pl
from jax.experimental.pallas import tpu as pltpu
from jax.experimental.pallas import tpu_sc as plsc

DIM = 4096
N_TOKENS = 8192
L = 16
NC = 2
NS = 16
NW = NC * NS
ROWS_PER_W = N_TOKENS // NW
R = 4
N_CHUNKS = ROWS_PER_W // R
NBLK = DIM // L


def _permute_body(x_hbm, p_hbm, out_hbm, p_v, x0, x1, o0, o1,
                  si0, si1, so0, so1):
    c = lax.axis_index("c")
    s = lax.axis_index("s")
    wid = s * NC + c
    row0 = wid * ROWS_PER_W

    xb = (x0, x1)
    ob = (o0, o1)
    sin = (si0, si1)
    sout = (so0, so1)

    pltpu.sync_copy(p_hbm, p_v)

    def in_copy(g, b):
        return pltpu.make_async_copy(
            x_hbm.at[pl.ds(row0 + g * R, R)], xb[b], sin[b])

    def out_copy(g, b):
        return pltpu.make_async_copy(
            ob[b], out_hbm.at[pl.ds(row0 + g * R, R)], sout[b])

    in_copy(0, 0).start()

    def loop(i, carry):
        for b in range(2):
            g = i * 2 + b

            @pl.when(g + 1 < N_CHUNKS)
            def _():
                in_copy(g + 1, 1 - b).start()

            in_copy(g, b).wait()

            @pl.when(g >= 2)
            def _():
                out_copy(g - 2, b).wait()

            x_v = xb[b]
            o_v = ob[b]

            @plsc.parallel_loop(0, NBLK, unroll=4)
            def blk(jb):
                idx = p_v[pl.ds(jb * L, L)]
                for r in range(R):
                    ridx = jnp.full((L,), r, dtype=jnp.int32)
                    vals = plsc.load_gather(x_v, [ridx, idx])
                    o_v[r, pl.ds(jb * L, L)] = vals

            out_copy(g, b).start()
        return carry

    lax.fori_loop(0, N_CHUNKS // 2, loop, 0)

    for b in range(2):
        out_copy(N_CHUNKS - 2 + b, b).wait()


@jax.jit
def _permute_in(x, p32):
    mesh = plsc.VectorSubcoreMesh(core_axis_name="c", subcore_axis_name="s")
    f = functools.partial(
        pl.kernel,
        out_type=jax.ShapeDtypeStruct((N_TOKENS, DIM), jnp.float32),
        mesh=mesh,
        scratch_types=[
            pltpu.VMEM((DIM,), jnp.int32),
            pltpu.VMEM((R, DIM), jnp.float32),
            pltpu.VMEM((R, DIM), jnp.float32),
            pltpu.VMEM((R, DIM), jnp.float32),
            pltpu.VMEM((R, DIM), jnp.float32),
            pltpu.SemaphoreType.DMA,
            pltpu.SemaphoreType.DMA,
            pltpu.SemaphoreType.DMA,
            pltpu.SemaphoreType.DMA,
        ],
        compiler_params=pltpu.CompilerParams(
            use_tc_tiling_on_sc=True, needs_layout_passes=False
        ),
    )(_permute_body)
    return f(x, p32)


def kernel(x, permute):
    return _permute_in(x, permute.astype(jnp.int32))


# speedup vs baseline: 2.9264x; 1.0058x over previous
import functools

import jax
import jax.numpy as jnp
from jax import lax
from jax.experimental import pallas as pl
from jax.experimental.pallas import tpu as pltpu
from jax.experimental.pallas import tpu_sc as plsc

DIM = 4096
N_TOKENS = 8192
L = 16
NC = 2
NS = 16
NW = NC * NS
ROWS_PER_W = N_TOKENS // NW
R = 4
N_CHUNKS = ROWS_PER_W // R
NBLK = DIM // L


def _permute_body(x_hbm, p_hbm, out_hbm, p_v, x0, x1, o0, o1,
                  si0, si1, so0, so1):
    c = lax.axis_index("c")
    s = lax.axis_index("s")
    wid = s * NC + c
    row0 = wid * ROWS_PER_W

    xb = (x0, x1)
    ob = (o0, o1)
    sin = (si0, si1)
    sout = (so0, so1)

    pltpu.sync_copy(p_hbm, p_v)

    def in_copy(g, b):
        return pltpu.make_async_copy(
            x_hbm.at[pl.ds(row0 + g * R, R)], xb[b], sin[b])

    def out_copy(g, b):
        return pltpu.make_async_copy(
            ob[b], out_hbm.at[pl.ds(row0 + g * R, R)], sout[b])

    in_copy(0, 0).start()

    def loop(i, carry):
        for b in range(2):
            g = i * 2 + b

            @pl.when(g + 1 < N_CHUNKS)
            def _():
                in_copy(g + 1, 1 - b).start()

            in_copy(g, b).wait()

            @pl.when(g >= 2)
            def _():
                out_copy(g - 2, b).wait()

            x_v = xb[b]
            o_v = ob[b]

            @plsc.parallel_loop(0, NBLK, unroll=8)
            def blk(jb):
                idx = p_v[pl.ds(jb * L, L)]
                for r in range(R):
                    ridx = jnp.full((L,), r, dtype=jnp.int32)
                    vals = plsc.load_gather(x_v, [ridx, idx])
                    o_v[r, pl.ds(jb * L, L)] = vals

            out_copy(g, b).start()
        return carry

    lax.fori_loop(0, N_CHUNKS // 2, loop, 0)

    for b in range(2):
        out_copy(N_CHUNKS - 2 + b, b).wait()


@jax.jit
def _permute_in(x, p32):
    mesh = plsc.VectorSubcoreMesh(core_axis_name="c", subcore_axis_name="s")
    f = functools.partial(
        pl.kernel,
        out_type=jax.ShapeDtypeStruct((N_TOKENS, DIM), jnp.float32),
        mesh=mesh,
        scratch_types=[
            pltpu.VMEM((DIM,), jnp.int32),
            pltpu.VMEM((R, DIM), jnp.float32),
            pltpu.VMEM((R, DIM), jnp.float32),
            pltpu.VMEM((R, DIM), jnp.float32),
            pltpu.VMEM((R, DIM), jnp.float32),
            pltpu.SemaphoreType.DMA,
            pltpu.SemaphoreType.DMA,
            pltpu.SemaphoreType.DMA,
            pltpu.SemaphoreType.DMA,
        ],
        compiler_params=pltpu.CompilerParams(
            use_tc_tiling_on_sc=True, needs_layout_passes=False
        ),
    )(_permute_body)
    return f(x, p32)


def kernel(x, permute):
    return _permute_in(x, permute.astype(jnp.int32))


# speedup vs baseline: 3.0031x; 1.0262x over previous
import functools

import jax
import jax.numpy as jnp
from jax import lax
from jax.experimental import pallas as pl
from jax.experimental.pallas import tpu as pltpu
from jax.experimental.pallas import tpu_sc as plsc

DIM = 4096
N_TOKENS = 8192
L = 16
NC = 2
NS = 16
NW = NC * NS
ROWS_PER_W = N_TOKENS // NW
R = 8
HR = R // 2
N_CHUNKS = ROWS_PER_W // R
NBLK = DIM // L


def _permute_body(x_hbm, p_hbm, out_hbm, p_v, x0, x1, o0, o1,
                  si0, si1, so0, so1):
    c = lax.axis_index("c")
    s = lax.axis_index("s")
    wid = s * NC + c
    row0 = wid * ROWS_PER_W

    xb = (x0, x1)
    ob = (o0, o1)
    sin = (si0, si1)
    sout = (so0, so1)

    pltpu.sync_copy(p_hbm, p_v)

    def in_copy(g, b):
        return pltpu.make_async_copy(
            x_hbm.at[pl.ds(row0 + g * R, R)], xb[b], sin[b])

    def out_copy(g, h):
        return pltpu.make_async_copy(
            ob[h], out_hbm.at[pl.ds(row0 + g * R + h * HR, HR)], sout[h])

    in_copy(0, 0).start()

    def chunk_loop(i, carry):
        for b in range(2):
            g = i * 2 + b

            @pl.when(g + 1 < N_CHUNKS)
            def _():
                in_copy(g + 1, 1 - b).start()

            in_copy(g, b).wait()

            x_v = xb[b]

            for h in range(2):
                @pl.when(g >= 1)
                def _():
                    out_copy(g - 1, h).wait()

                o_v = ob[h]

                @plsc.parallel_loop(0, NBLK, unroll=8)
                def blk(jb):
                    idx = p_v[pl.ds(jb * L, L)]
                    for r in range(HR):
                        ridx = jnp.full((L,), h * HR + r, dtype=jnp.int32)
                        vals = plsc.load_gather(x_v, [ridx, idx])
                        o_v[r, pl.ds(jb * L, L)] = vals

                out_copy(g, h).start()
        return carry

    lax.fori_loop(0, N_CHUNKS // 2, chunk_loop, 0)

    for h in range(2):
        out_copy(N_CHUNKS - 1, h).wait()


@jax.jit
def _permute_in(x, p32):
    mesh = plsc.VectorSubcoreMesh(core_axis_name="c", subcore_axis_name="s")
    f = functools.partial(
        pl.kernel,
        out_type=jax.ShapeDtypeStruct((N_TOKENS, DIM), jnp.float32),
        mesh=mesh,
        scratch_types=[
            pltpu.VMEM((DIM,), jnp.int32),
            pltpu.VMEM((R, DIM), jnp.float32),
            pltpu.VMEM((R, DIM), jnp.float32),
            pltpu.VMEM((HR, DIM), jnp.float32),
            pltpu.VMEM((HR, DIM), jnp.float32),
            pltpu.SemaphoreType.DMA,
            pltpu.SemaphoreType.DMA,
            pltpu.SemaphoreType.DMA,
            pltpu.SemaphoreType.DMA,
        ],
        compiler_params=pltpu.CompilerParams(
            use_tc_tiling_on_sc=True, needs_layout_passes=False
        ),
    )(_permute_body)
    return f(x, p32)


def kernel(x, permute):
    return _permute_in(x, permute.astype(jnp.int32))


# speedup vs baseline: 3.0061x; 1.0010x over previous
import functools

import jax
import jax.numpy as jnp
from jax import lax
from jax.experimental import pallas as pl
from jax.experimental.pallas import tpu as pltpu
from jax.experimental.pallas import tpu_sc as plsc

DIM = 4096
N_TOKENS = 8192
L = 16
NC = 2
NS = 16
NW = NC * NS
ROWS_PER_W = N_TOKENS // NW
R = 8
HR = R // 2
N_CHUNKS = ROWS_PER_W // R
NBLK = DIM // L


def _permute_body(x_hbm, p_hbm, out_hbm, p_v, x0, x1, o0, o1,
                  si00, si01, si10, si11, so0, so1):
    c = lax.axis_index("c")
    s = lax.axis_index("s")
    wid = s * NC + c
    row0 = wid * ROWS_PER_W

    xb = (x0, x1)
    ob = (o0, o1)
    sin = ((si00, si01), (si10, si11))
    sout = (so0, so1)

    pltpu.sync_copy(p_hbm, p_v)

    def in_copy(g, b, h):
        return pltpu.make_async_copy(
            x_hbm.at[pl.ds(row0 + g * R + h * HR, HR)],
            xb[b].at[pl.ds(h * HR, HR)],
            sin[b][h])

    def out_copy(g, h):
        return pltpu.make_async_copy(
            ob[h], out_hbm.at[pl.ds(row0 + g * R + h * HR, HR)], sout[h])

    in_copy(0, 0, 0).start()
    in_copy(0, 0, 1).start()

    def chunk_loop(i, carry):
        for b in range(2):
            g = i * 2 + b

            @pl.when(g + 1 < N_CHUNKS)
            def _():
                in_copy(g + 1, 1 - b, 0).start()
                in_copy(g + 1, 1 - b, 1).start()

            x_v = xb[b]

            for h in range(2):
                in_copy(g, b, h).wait()

                @pl.when(g >= 1)
                def _():
                    out_copy(g - 1, h).wait()

                o_v = ob[h]

                @plsc.parallel_loop(0, NBLK, unroll=8)
                def blk(jb):
                    idx = p_v[pl.ds(jb * L, L)]
                    for r in range(HR):
                        ridx = jnp.full((L,), h * HR + r, dtype=jnp.int32)
                        vals = plsc.load_gather(x_v, [ridx, idx])
                        o_v[r, pl.ds(jb * L, L)] = vals

                out_copy(g, h).start()
        return carry

    lax.fori_loop(0, N_CHUNKS // 2, chunk_loop, 0)

    for h in range(2):
        out_copy(N_CHUNKS - 1, h).wait()


@jax.jit
def _permute_in(x, p32):
    mesh = plsc.VectorSubcoreMesh(core_axis_name="c", subcore_axis_name="s")
    f = functools.partial(
        pl.kernel,
        out_type=jax.ShapeDtypeStruct((N_TOKENS, DIM), jnp.float32),
        mesh=mesh,
        scratch_types=[
            pltpu.VMEM((DIM,), jnp.int32),
            pltpu.VMEM((R, DIM), jnp.float32),
            pltpu.VMEM((R, DIM), jnp.float32),
            pltpu.VMEM((HR, DIM), jnp.float32),
            pltpu.VMEM((HR, DIM), jnp.float32),
            pltpu.SemaphoreType.DMA,
            pltpu.SemaphoreType.DMA,
            pltpu.SemaphoreType.DMA,
            pltpu.SemaphoreType.DMA,
            pltpu.SemaphoreType.DMA,
            pltpu.SemaphoreType.DMA,
        ],
        compiler_params=pltpu.CompilerParams(
            use_tc_tiling_on_sc=True, needs_layout_passes=False
        ),
    )(_permute_body)
    return f(x, p32)


def kernel(x, permute):
    return _permute_in(x, permute.astype(jnp.int32))


# speedup vs baseline: 3.0124x; 1.0021x over previous
import functools

import jax
import jax.numpy as jnp
from jax import lax
from jax.experimental import pallas as pl
from jax.experimental.pallas import tpu as pltpu
from jax.experimental.pallas import tpu_sc as plsc

DIM = 4096
N_TOKENS = 8192
L = 16
NC = 2
NS = 16
NW = NC * NS
ROWS_PER_W = N_TOKENS // NW
R = 8
HR = R // 2
N_CHUNKS = ROWS_PER_W // R
NBLK = DIM // L


def _permute_body(x_hbm, p_hbm, out_hbm, p_v, x0, x1, o0, o1,
                  si0, si1, so0, so1):
    c = lax.axis_index("c")
    s = lax.axis_index("s")
    wid = s * NC + c
    row0 = wid * ROWS_PER_W

    xb = (x0, x1)
    ob = (o0, o1)
    sin = (si0, si1)
    sout = (so0, so1)

    pltpu.sync_copy(p_hbm, p_v)

    def in_copy(g, b):
        return pltpu.make_async_copy(
            x_hbm.at[pl.ds(row0 + g * R, R)], xb[b], sin[b])

    def out_copy(g, h):
        return pltpu.make_async_copy(
            ob[h], out_hbm.at[pl.ds(row0 + g * R + h * HR, HR)], sout[h])

    in_copy(0, 0).start()

    def chunk_loop(i, carry):
        for b in range(2):
            g = i * 2 + b

            @pl.when(g + 1 < N_CHUNKS)
            def _():
                in_copy(g + 1, 1 - b).start()

            in_copy(g, b).wait()

            x_v = xb[b]

            for h in range(2):
                @pl.when(g >= 1)
                def _():
                    out_copy(g - 1, h).wait()

                o_v = ob[h]

                @plsc.parallel_loop(0, NBLK, unroll=8)
                def blk(jb):
                    idx = p_v[pl.ds(jb * L, L)]
                    for r in range(HR):
                        ridx = jnp.full((L,), h * HR + r, dtype=jnp.int32)
                        vals = plsc.load_gather(x_v, [ridx, idx])
                        o_v[r, pl.ds(jb * L, L)] = vals

                out_copy(g, h).start()
        return carry

    lax.fori_loop(0, N_CHUNKS // 2, chunk_loop, 0)

    for h in range(2):
        out_copy(N_CHUNKS - 1, h).wait()


@jax.jit
def _permute_in(x, p32):
    mesh = plsc.VectorSubcoreMesh(core_axis_name="c", subcore_axis_name="s")
    f = functools.partial(
        pl.kernel,
        out_type=jax.ShapeDtypeStruct((N_TOKENS, DIM), jnp.float32),
        mesh=mesh,
        scratch_types=[
            pltpu.VMEM((DIM,), jnp.int32),
            pltpu.VMEM((R, DIM), jnp.float32),
            pltpu.VMEM((R, DIM), jnp.float32),
            pltpu.VMEM((HR, DIM), jnp.float32),
            pltpu.VMEM((HR, DIM), jnp.float32),
            pltpu.SemaphoreType.DMA,
            pltpu.SemaphoreType.DMA,
            pltpu.SemaphoreType.DMA,
            pltpu.SemaphoreType.DMA,
        ],
        compiler_params=pltpu.CompilerParams(
            use_tc_tiling_on_sc=True, needs_layout_passes=False
        ),
    )(_permute_body)
    return f(x, p32)


def kernel(x, permute):
    return _permute_in(x, permute.astype(jnp.int32))
